# initial kernel scaffold (unmeasured)
import jax
import jax.numpy as jnp
from jax import lax
from jax.experimental import pallas as pl
from jax.experimental.pallas import tpu as pltpu

N_DEV = 4
N_LAYERS = 3
N_HOPS = (1 + N_LAYERS) * (N_DEV - 1)


def kernel(x, Win0, Wout0, Win1, Wout1, Win2, Wout2):
    m_per, d = x.shape
    M = N_DEV * m_per

    def body(x_ref, win0, wout0, win1, wout1, win2, wout2,
             out_ref, ag_comm, ar_comm, send_sems, recv_sems):
        my = lax.axis_index("i")
        left = (my - 1) % N_DEV
        right = (my + 1) % N_DEV

        barrier_sem = pltpu.get_barrier_semaphore()
        for nbr in (left, right):
            pl.semaphore_signal(barrier_sem, inc=1, device_id=(nbr,),
                                device_id_type=pl.DeviceIdType.MESH)
        pl.semaphore_wait(barrier_sem, 2)

        hop = 0

        def ring_hop(comm, slot_s, slot_r):
            nonlocal hop
            rdma = pltpu.make_async_remote_copy(
                src_ref=comm.at[slot_s],
                dst_ref=comm.at[slot_r],
                send_sem=send_sems.at[hop],
                recv_sem=recv_sems.at[hop],
                device_id=(right,),
                device_id_type=pl.DeviceIdType.MESH,
            )
            rdma.start()
            rdma.wait()
            hop += 1

        out_ref[pl.ds(my * m_per, m_per), :] = x_ref[...]
        ag_comm[0] = x_ref[...]
        for h in range(N_DEV - 1):
            s, r = h % 2, (h + 1) % 2
            ring_hop(ag_comm, s, r)
            origin = (my - h - 1) % N_DEV
            out_ref[pl.ds(origin * m_per, m_per), :] = ag_comm[r]

        for win, wout in ((win0, wout0), (win1, wout1), (win2, wout2)):
            hact = jnp.maximum(
                jnp.dot(out_ref[...], win[...],
                        preferred_element_type=jnp.float32,
                        precision=lax.Precision.HIGHEST),
                0.0,
            )
            part = jnp.dot(hact, wout[...],
                           preferred_element_type=jnp.float32,
                           precision=lax.Precision.HIGHEST)
            ar_comm[0] = part
            out_ref[...] = part
            for h in range(N_DEV - 1):
                s, r = h % 2, (h + 1) % 2
                ring_hop(ar_comm, s, r)
                out_ref[...] += ar_comm[r]

    return pl.pallas_call(
        body,
        out_shape=jax.ShapeDtypeStruct((M, d), jnp.float32),
        in_specs=[pl.BlockSpec(memory_space=pltpu.VMEM)] * 7,
        out_specs=pl.BlockSpec(memory_space=pltpu.VMEM),
        scratch_shapes=[
            pltpu.VMEM((2, m_per, d), jnp.float32),
            pltpu.VMEM((2, M, d), jnp.float32),
            pltpu.SemaphoreType.DMA((N_HOPS,)),
            pltpu.SemaphoreType.DMA((N_HOPS,)),
        ],
        compiler_params=pltpu.CompilerParams(collective_id=0),
    )(x, Win0, Wout0, Win1, Wout1, Win2, Wout2)


# baseline (device time: 197468 ns/iter reference)
import jax
import jax.numpy as jnp
from jax import lax
from jax.experimental import pallas as pl
from jax.experimental.pallas import tpu as pltpu

N_DEV = 4
N_LAYERS = 3
N_HOPS = (1 + N_LAYERS) * (N_DEV - 1)


def kernel(x, Win0, Wout0, Win1, Wout1, Win2, Wout2):
    m_per, d = x.shape
    M = N_DEV * m_per

    def body(x_ref, win0, wout0, win1, wout1, win2, wout2,
             out_ref, ag_comm, ar_comm, send_sems, recv_sems):
        my = lax.axis_index("i")
        left = (my - 1) % N_DEV
        right = (my + 1) % N_DEV

        barrier_sem = pltpu.get_barrier_semaphore()
        for nbr in (left, right):
            pl.semaphore_signal(barrier_sem, inc=1, device_id=(nbr,),
                                device_id_type=pl.DeviceIdType.MESH)
        pl.semaphore_wait(barrier_sem, 2)

        hop = 0

        def ring_hop(comm, slot_s, slot_r):
            nonlocal hop
            rdma = pltpu.make_async_remote_copy(
                src_ref=comm.at[slot_s],
                dst_ref=comm.at[slot_r],
                send_sem=send_sems.at[hop],
                recv_sem=recv_sems.at[hop],
                device_id=(right,),
                device_id_type=pl.DeviceIdType.MESH,
            )
            rdma.start()
            rdma.wait()
            hop += 1

        out_ref[pl.ds(my * m_per, m_per), :] = x_ref[...]
        ag_comm[0] = x_ref[...]
        for h in range(N_DEV - 1):
            s, r = h % 2, (h + 1) % 2
            ring_hop(ag_comm, s, r)
            origin = (my - h - 1) % N_DEV
            out_ref[pl.ds(origin * m_per, m_per), :] = ag_comm[r]

        for win, wout in ((win0, wout0), (win1, wout1), (win2, wout2)):
            hact = jnp.maximum(
                jnp.dot(out_ref[...], win[...],
                        preferred_element_type=jnp.float32,
                        precision=lax.Precision.HIGHEST),
                0.0,
            )
            part = jnp.dot(hact, wout[...],
                           preferred_element_type=jnp.float32,
                           precision=lax.Precision.HIGHEST)
            ar_comm[0] = part
            out_ref[...] = part
            for h in range(N_DEV - 1):
                s, r = h % 2, (h + 1) % 2
                ring_hop(ar_comm, s, r)
                out_ref[...] += ar_comm[r]

    return pl.pallas_call(
        body,
        out_shape=jax.ShapeDtypeStruct((M, d), jnp.float32),
        in_specs=[pl.BlockSpec(memory_space=pltpu.VMEM)] * 7,
        out_specs=pl.BlockSpec(memory_space=pltpu.VMEM),
        scratch_shapes=[
            pltpu.VMEM((2, m_per, d), jnp.float32),
            pltpu.VMEM((2, M, d), jnp.float32),
            pltpu.SemaphoreType.DMA((N_HOPS,)),
            pltpu.SemaphoreType.DMA((N_HOPS,)),
        ],
        compiler_params=pltpu.CompilerParams(
            collective_id=0,
            vmem_limit_bytes=100 * 1024 * 1024,
        ),
    )(x, Win0, Wout0, Win1, Wout1, Win2, Wout2)


# device time: 160474 ns/iter; 1.2305x vs baseline; 1.2305x over previous
import jax
import jax.numpy as jnp
from jax import lax
from jax.experimental import pallas as pl
from jax.experimental.pallas import tpu as pltpu

N_DEV = 4
N_LAYERS = 3
N_HOPS = (1 + N_LAYERS) * (N_DEV - 1)


def kernel(x, Win0, Wout0, Win1, Wout1, Win2, Wout2):
    m_per, d = x.shape
    M = N_DEV * m_per

    def body(x_ref, win0, wout0, win1, wout1, win2, wout2,
             out_ref, ag_comm, ar_comm, send_sems, recv_sems):
        my = lax.axis_index("i")
        left = (my - 1) % N_DEV
        right = (my + 1) % N_DEV

        barrier_sem = pltpu.get_barrier_semaphore()
        for nbr in (left, right):
            pl.semaphore_signal(barrier_sem, inc=1, device_id=(nbr,),
                                device_id_type=pl.DeviceIdType.MESH)
        pl.semaphore_wait(barrier_sem, 2)

        hop = 0

        def ring_hop(comm, slot_s, slot_r):
            nonlocal hop
            rdma = pltpu.make_async_remote_copy(
                src_ref=comm.at[slot_s],
                dst_ref=comm.at[slot_r],
                send_sem=send_sems.at[hop],
                recv_sem=recv_sems.at[hop],
                device_id=(right,),
                device_id_type=pl.DeviceIdType.MESH,
            )
            rdma.start()
            rdma.wait()
            hop += 1

        out_ref[pl.ds(my * m_per, m_per), :] = x_ref[...]
        ag_comm[0] = x_ref[...]
        for h in range(N_DEV - 1):
            s, r = h % 2, (h + 1) % 2
            ring_hop(ag_comm, s, r)
            origin = (my - h - 1) % N_DEV
            out_ref[pl.ds(origin * m_per, m_per), :] = ag_comm[r]

        for win, wout in ((win0, wout0), (win1, wout1), (win2, wout2)):
            hact = jnp.maximum(
                jnp.dot(out_ref[...], win[...],
                        preferred_element_type=jnp.float32),
                0.0,
            )
            part = jnp.dot(hact, wout[...],
                           preferred_element_type=jnp.float32)
            ar_comm[0] = part
            out_ref[...] = part
            for h in range(N_DEV - 1):
                s, r = h % 2, (h + 1) % 2
                ring_hop(ar_comm, s, r)
                out_ref[...] += ar_comm[r]

    return pl.pallas_call(
        body,
        out_shape=jax.ShapeDtypeStruct((M, d), jnp.float32),
        in_specs=[pl.BlockSpec(memory_space=pltpu.VMEM)] * 7,
        out_specs=pl.BlockSpec(memory_space=pltpu.VMEM),
        scratch_shapes=[
            pltpu.VMEM((2, m_per, d), jnp.float32),
            pltpu.VMEM((2, M, d), jnp.float32),
            pltpu.SemaphoreType.DMA((N_HOPS,)),
            pltpu.SemaphoreType.DMA((N_HOPS,)),
        ],
        compiler_params=pltpu.CompilerParams(
            collective_id=0,
            vmem_limit_bytes=100 * 1024 * 1024,
        ),
    )(x, Win0, Wout0, Win1, Wout1, Win2, Wout2)


# device time: 84430 ns/iter; 2.3388x vs baseline; 1.9007x over previous
import jax
import jax.numpy as jnp
from jax import lax
from jax.experimental import pallas as pl
from jax.experimental.pallas import tpu as pltpu

N_DEV = 4
N_LAYERS = 3
N_PHASES = 1 + 2 * N_LAYERS


def kernel(x, Win0, Wout0, Win1, Wout1, Win2, Wout2):
    m_per, d = x.shape
    M = N_DEV * m_per

    def body(x_ref, win0, wout0, win1, wout1, win2, wout2, out_ref,
             acts, rs_recv, psbuf, send_sems, recv_sems):
        my = lax.axis_index("i")

        barrier_sem = pltpu.get_barrier_semaphore()
        for o in range(1, N_DEV):
            pl.semaphore_signal(barrier_sem, inc=1,
                                device_id=((my + o) % N_DEV,),
                                device_id_type=pl.DeviceIdType.MESH)
        pl.semaphore_wait(barrier_sem, N_DEV - 1)

        def send_to_all(phase, src_for, dst_self):
            rds = []
            for o in range(1, N_DEV):
                t = (my + o) % N_DEV
                rdma = pltpu.make_async_remote_copy(
                    src_ref=src_for(t),
                    dst_ref=dst_self,
                    send_sem=send_sems.at[phase, o - 1],
                    recv_sem=recv_sems.at[phase, my],
                    device_id=(t,),
                    device_id_type=pl.DeviceIdType.MESH,
                )
                rdma.start()
                rds.append(rdma)
            return rds

        def wait_recvs(phase, dst_for_origin):
            for o in range(1, N_DEV):
                s = (my + o) % N_DEV
                rdma = pltpu.make_async_remote_copy(
                    src_ref=dst_for_origin(s),
                    dst_ref=dst_for_origin(s),
                    send_sem=send_sems.at[phase, o - 1],
                    recv_sem=recv_sems.at[phase, s],
                    device_id=(s,),
                    device_id_type=pl.DeviceIdType.MESH,
                )
                rdma.wait_recv()

        acts[0, pl.ds(my * m_per, m_per), :] = x_ref[...]
        sends = send_to_all(
            0,
            lambda t: x_ref,
            acts.at[0, pl.ds(my * m_per, m_per), :],
        )
        wait_recvs(0, lambda s: acts.at[0, pl.ds(s * m_per, m_per), :])
        for r in sends:
            r.wait_send()

        weights = ((win0, wout0), (win1, wout1), (win2, wout2))
        for l, (win, wout) in enumerate(weights):
            p_rs, p_ag = 1 + 2 * l, 2 + 2 * l

            hact = jnp.maximum(
                jnp.dot(acts[l], win[...],
                        preferred_element_type=jnp.float32),
                0.0,
            )
            psbuf[...] = jnp.dot(hact, wout[...],
                                 preferred_element_type=jnp.float32)

            sends = send_to_all(
                p_rs,
                lambda t: psbuf.at[pl.ds(t * m_per, m_per), :],
                rs_recv.at[l, my],
            )
            wait_recvs(p_rs, lambda s: rs_recv.at[l, s])
            for r in sends:
                r.wait_send()

            psum = psbuf[pl.ds(my * m_per, m_per), :]
            for o in range(1, N_DEV):
                s = (my + o) % N_DEV
                psum = psum + rs_recv[l, s]

            if l < N_LAYERS - 1:
                dst = acts.at[l + 1]
            else:
                dst = out_ref
            dst[pl.ds(my * m_per, m_per), :] = psum
            sends = send_to_all(
                p_ag,
                lambda t: dst.at[pl.ds(my * m_per, m_per), :],
                dst.at[pl.ds(my * m_per, m_per), :],
            )
            wait_recvs(p_ag, lambda s: dst.at[pl.ds(s * m_per, m_per), :])
            for r in sends:
                r.wait_send()

    return pl.pallas_call(
        body,
        out_shape=jax.ShapeDtypeStruct((M, d), jnp.float32),
        in_specs=[pl.BlockSpec(memory_space=pltpu.VMEM)] * 7,
        out_specs=pl.BlockSpec(memory_space=pltpu.VMEM),
        scratch_shapes=[
            pltpu.VMEM((N_LAYERS, M, d), jnp.float32),
            pltpu.VMEM((N_LAYERS, N_DEV, m_per, d), jnp.float32),
            pltpu.VMEM((M, d), jnp.float32),
            pltpu.SemaphoreType.DMA((N_PHASES, N_DEV - 1)),
            pltpu.SemaphoreType.DMA((N_PHASES, N_DEV)),
        ],
        compiler_params=pltpu.CompilerParams(
            collective_id=0,
            vmem_limit_bytes=100 * 1024 * 1024,
        ),
    )(x, Win0, Wout0, Win1, Wout1, Win2, Wout2)


# device time: 67871 ns/iter; 2.9095x vs baseline; 1.2440x over previous
import jax
import jax.numpy as jnp
from jax import lax
from jax.experimental import pallas as pl
from jax.experimental.pallas import tpu as pltpu

N_DEV = 4
N_LAYERS = 3
N_PHASES = 1 + 2 * N_LAYERS


def kernel(x, Win0, Wout0, Win1, Wout1, Win2, Wout2):
    m_per, d = x.shape
    M = N_DEV * m_per

    def body(x_ref, win0, wout0, win1, wout1, win2, wout2, out_ref,
             acts, rs_recv, psbuf, send_sems, recv_sems):
        my = lax.axis_index("i")

        barrier_sem = pltpu.get_barrier_semaphore()
        for o in range(1, N_DEV):
            pl.semaphore_signal(barrier_sem, inc=1,
                                device_id=((my + o) % N_DEV,),
                                device_id_type=pl.DeviceIdType.MESH)
        pl.semaphore_wait(barrier_sem, N_DEV - 1)

        def send_to_all(phase, src_for, dst_self):
            rds = []
            for o in range(1, N_DEV):
                t = (my + o) % N_DEV
                rdma = pltpu.make_async_remote_copy(
                    src_ref=src_for(t),
                    dst_ref=dst_self,
                    send_sem=send_sems.at[phase, o - 1],
                    recv_sem=recv_sems.at[phase, my],
                    device_id=(t,),
                    device_id_type=pl.DeviceIdType.MESH,
                )
                rdma.start()
                rds.append(rdma)
            return rds

        def wait_recvs(phase, dst_for_origin):
            for o in range(1, N_DEV):
                s = (my + o) % N_DEV
                rdma = pltpu.make_async_remote_copy(
                    src_ref=dst_for_origin(s),
                    dst_ref=dst_for_origin(s),
                    send_sem=send_sems.at[phase, o - 1],
                    recv_sem=recv_sems.at[phase, s],
                    device_id=(s,),
                    device_id_type=pl.DeviceIdType.MESH,
                )
                rdma.wait_recv()

        acts[0, pl.ds(my * m_per, m_per), :] = x_ref[...].astype(jnp.bfloat16)
        sends = send_to_all(
            0,
            lambda t: acts.at[0, pl.ds(my * m_per, m_per), :],
            acts.at[0, pl.ds(my * m_per, m_per), :],
        )
        wait_recvs(0, lambda s: acts.at[0, pl.ds(s * m_per, m_per), :])
        for r in sends:
            r.wait_send()

        weights = ((win0, wout0), (win1, wout1), (win2, wout2))
        for l, (win, wout) in enumerate(weights):
            p_rs, p_ag = 1 + 2 * l, 2 + 2 * l
            last = l == N_LAYERS - 1

            hact = jnp.maximum(
                jnp.dot(acts[l], win[...],
                        preferred_element_type=jnp.float32),
                0.0,
            )
            psbuf[...] = jnp.dot(
                hact, wout[...], preferred_element_type=jnp.float32
            ).astype(jnp.bfloat16)

            sends = send_to_all(
                p_rs,
                lambda t: psbuf.at[pl.ds(t * m_per, m_per), :],
                rs_recv.at[l, my],
            )
            wait_recvs(p_rs, lambda s: rs_recv.at[l, s])
            for r in sends:
                r.wait_send()

            psum = psbuf[pl.ds(my * m_per, m_per), :].astype(jnp.float32)
            for o in range(1, N_DEV):
                s = (my + o) % N_DEV
                psum = psum + rs_recv[l, s].astype(jnp.float32)

            if not last:
                dst = acts.at[l + 1]
                dst[pl.ds(my * m_per, m_per), :] = psum.astype(jnp.bfloat16)
            else:
                dst = out_ref
                dst[pl.ds(my * m_per, m_per), :] = psum
            sends = send_to_all(
                p_ag,
                lambda t: dst.at[pl.ds(my * m_per, m_per), :],
                dst.at[pl.ds(my * m_per, m_per), :],
            )
            wait_recvs(p_ag, lambda s: dst.at[pl.ds(s * m_per, m_per), :])
            for r in sends:
                r.wait_send()

    return pl.pallas_call(
        body,
        out_shape=jax.ShapeDtypeStruct((M, d), jnp.float32),
        in_specs=[pl.BlockSpec(memory_space=pltpu.VMEM)] * 7,
        out_specs=pl.BlockSpec(memory_space=pltpu.VMEM),
        scratch_shapes=[
            pltpu.VMEM((N_LAYERS, M, d), jnp.bfloat16),
            pltpu.VMEM((N_LAYERS, N_DEV, m_per, d), jnp.bfloat16),
            pltpu.VMEM((M, d), jnp.bfloat16),
            pltpu.SemaphoreType.DMA((N_PHASES, N_DEV - 1)),
            pltpu.SemaphoreType.DMA((N_PHASES, N_DEV)),
        ],
        compiler_params=pltpu.CompilerParams(
            collective_id=0,
            vmem_limit_bytes=100 * 1024 * 1024,
        ),
    )(x, Win0, Wout0, Win1, Wout1, Win2, Wout2)


# device time: 66095 ns/iter; 2.9876x vs baseline; 1.0269x over previous
import jax
import jax.numpy as jnp
from jax import lax
from jax.experimental import pallas as pl
from jax.experimental.pallas import tpu as pltpu

N_DEV = 4
N_LAYERS = 3
N_PHASES = 1 + 2 * N_LAYERS


def kernel(x, Win0, Wout0, Win1, Wout1, Win2, Wout2):
    m_per, d = x.shape
    M = N_DEV * m_per

    def body(x_ref, win0, wout0, win1, wout1, win2, wout2, out_ref,
             acts, rs_recv, psbuf, send_sems, recv_sems):
        my = lax.axis_index("i")

        barrier_sem = pltpu.get_barrier_semaphore()
        for o in range(1, N_DEV):
            pl.semaphore_signal(barrier_sem, inc=1,
                                device_id=((my + o) % N_DEV,),
                                device_id_type=pl.DeviceIdType.MESH)
        pl.semaphore_wait(barrier_sem, N_DEV - 1)

        def send_one(phase, o, src, dst_self):
            t = (my + o) % N_DEV
            rdma = pltpu.make_async_remote_copy(
                src_ref=src,
                dst_ref=dst_self,
                send_sem=send_sems.at[phase, o - 1],
                recv_sem=recv_sems.at[phase, my],
                device_id=(t,),
                device_id_type=pl.DeviceIdType.MESH,
            )
            rdma.start()
            return rdma

        def wait_recv_from(phase, o, dst):
            s = (my + o) % N_DEV
            rdma = pltpu.make_async_remote_copy(
                src_ref=dst,
                dst_ref=dst,
                send_sem=send_sems.at[phase, o - 1],
                recv_sem=recv_sems.at[phase, s],
                device_id=(s,),
                device_id_type=pl.DeviceIdType.MESH,
            )
            rdma.wait_recv()

        def rows(ref, idx):
            return ref.at[pl.ds(idx * m_per, m_per), :]

        AG_ORDER = (3, 1, 2)
        ARRIVE_ORDER = (1, 3, 2)

        acts[0, pl.ds(my * m_per, m_per), :] = x_ref[...].astype(jnp.bfloat16)
        pending = [
            send_one(0, o, rows(acts.at[0], my), rows(acts.at[0], my))
            for o in AG_ORDER
        ]

        weights = ((win0, wout0), (win1, wout1), (win2, wout2))
        for l, (win, wout) in enumerate(weights):
            p_prev, p_rs, p_ag = 2 * l, 2 * l + 1, 2 * l + 2
            last = l == N_LAYERS - 1

            def chunk(idx):
                xa = acts[l, pl.ds(idx * m_per, m_per), :]
                h = jnp.maximum(
                    jnp.dot(xa, win[...], preferred_element_type=jnp.float32),
                    0.0,
                )
                return jnp.dot(h, wout[...],
                               preferred_element_type=jnp.float32)

            psbuf[pl.ds(my * m_per, m_per), :] = chunk(my).astype(jnp.bfloat16)

            new_sends = []
            for o in ARRIVE_ORDER:
                s = (my + o) % N_DEV
                wait_recv_from(p_prev, o, rows(acts.at[l], s))
                psbuf[pl.ds(s * m_per, m_per), :] = chunk(s).astype(
                    jnp.bfloat16)
                new_sends.append(
                    send_one(p_rs, o, rows(psbuf, s), rs_recv.at[l, my]))

            for r in pending:
                r.wait_send()
            pending = new_sends

            for o in ARRIVE_ORDER:
                s = (my + o) % N_DEV
                wait_recv_from(p_rs, o, rs_recv.at[l, s])
            psum = psbuf[pl.ds(my * m_per, m_per), :].astype(jnp.float32)
            for o in range(1, N_DEV):
                s = (my + o) % N_DEV
                psum = psum + rs_recv[l, s].astype(jnp.float32)

            if not last:
                dst = acts.at[l + 1]
                dst[pl.ds(my * m_per, m_per), :] = psum.astype(jnp.bfloat16)
            else:
                dst = out_ref
                dst[pl.ds(my * m_per, m_per), :] = psum
            for o in AG_ORDER:
                pending.append(
                    send_one(p_ag, o, rows(dst, my), rows(dst, my)))

        for o in ARRIVE_ORDER:
            s = (my + o) % N_DEV
            wait_recv_from(2 * N_LAYERS, o, rows(out_ref, s))
        for r in pending:
            r.wait_send()

    return pl.pallas_call(
        body,
        out_shape=jax.ShapeDtypeStruct((M, d), jnp.float32),
        in_specs=[pl.BlockSpec(memory_space=pltpu.VMEM)] * 7,
        out_specs=pl.BlockSpec(memory_space=pltpu.VMEM),
        scratch_shapes=[
            pltpu.VMEM((N_LAYERS, M, d), jnp.bfloat16),
            pltpu.VMEM((N_LAYERS, N_DEV, m_per, d), jnp.bfloat16),
            pltpu.VMEM((M, d), jnp.bfloat16),
            pltpu.SemaphoreType.DMA((N_PHASES, N_DEV - 1)),
            pltpu.SemaphoreType.DMA((N_PHASES, N_DEV)),
        ],
        compiler_params=pltpu.CompilerParams(
            collective_id=0,
            vmem_limit_bytes=100 * 1024 * 1024,
        ),
    )(x, Win0, Wout0, Win1, Wout1, Win2, Wout2)


# device time: 53374 ns/iter; 3.6997x vs baseline; 1.2383x over previous
import jax
import jax.numpy as jnp
from jax import lax
from jax.experimental import pallas as pl
from jax.experimental.pallas import tpu as pltpu

N_DEV = 4
N_LAYERS = 3
N_PHASES = 1 + 2 * N_LAYERS


def kernel(x, Win0, Wout0, Win1, Wout1, Win2, Wout2):
    m_per, d = x.shape
    M = N_DEV * m_per

    def body(x_ref, win0, wout0, win1, wout1, win2, wout2, out_ref,
             acts, rs_recv, psbuf, win_vm, wout_vm, winb_vm, woutb_vm,
             outstage,
             send_sems, recv_sems, wcopy_sems, outcopy_sem):
        my = lax.axis_index("i")

        w_hbm = ((win0, wout0), (win1, wout1), (win2, wout2))

        def w_copies(l):
            return (
                pltpu.make_async_copy(w_hbm[l][0], win_vm.at[l % 2],
                                      wcopy_sems.at[l, 0]),
                pltpu.make_async_copy(w_hbm[l][1], wout_vm.at[l % 2],
                                      wcopy_sems.at[l, 1]),
            )

        for l in range(2):
            for c in w_copies(l):
                c.start()

        barrier_sem = pltpu.get_barrier_semaphore()
        for o in range(1, N_DEV):
            pl.semaphore_signal(barrier_sem, inc=1,
                                device_id=((my + o) % N_DEV,),
                                device_id_type=pl.DeviceIdType.MESH)
        pl.semaphore_wait(barrier_sem, N_DEV - 1)

        def send_one(phase, o, src, dst_self):
            t = (my + o) % N_DEV
            rdma = pltpu.make_async_remote_copy(
                src_ref=src,
                dst_ref=dst_self,
                send_sem=send_sems.at[phase, o - 1],
                recv_sem=recv_sems.at[phase, my],
                device_id=(t,),
                device_id_type=pl.DeviceIdType.MESH,
            )
            rdma.start()
            return rdma

        def wait_recv_from(phase, o, dst):
            s = (my + o) % N_DEV
            rdma = pltpu.make_async_remote_copy(
                src_ref=dst,
                dst_ref=dst,
                send_sem=send_sems.at[phase, o - 1],
                recv_sem=recv_sems.at[phase, s],
                device_id=(s,),
                device_id_type=pl.DeviceIdType.MESH,
            )
            rdma.wait_recv()

        def rows(ref, idx):
            return ref.at[pl.ds(idx * m_per, m_per), :]

        AG_ORDER = (3, 1, 2)
        ARRIVE_ORDER = (1, 3, 2)

        acts[0, pl.ds(my * m_per, m_per), :] = x_ref[...].astype(jnp.bfloat16)
        pending = [
            send_one(0, o, rows(acts.at[0], my), rows(acts.at[0], my))
            for o in AG_ORDER
        ]

        for c in w_copies(0):
            c.wait()
        winb_vm[0] = win_vm[0].astype(jnp.bfloat16)
        woutb_vm[0] = wout_vm[0].astype(jnp.bfloat16)
        for c in w_copies(2):
            c.start()

        for l in range(N_LAYERS):
            p_prev, p_rs, p_ag = 2 * l, 2 * l + 1, 2 * l + 2
            last = l == N_LAYERS - 1

            winb = winb_vm[l % 2]
            woutb = woutb_vm[l % 2]

            for o in ARRIVE_ORDER:
                s = (my + o) % N_DEV
                wait_recv_from(p_prev, o, rows(acts.at[l], s))
            h = jnp.maximum(
                jnp.dot(acts[l], winb,
                        preferred_element_type=jnp.float32),
                0.0,
            ).astype(jnp.bfloat16)
            psbuf[...] = jnp.dot(
                h, woutb, preferred_element_type=jnp.float32
            ).astype(jnp.bfloat16)

            new_sends = [
                send_one(p_rs, o, rows(psbuf, (my + o) % N_DEV),
                         rs_recv.at[l, my])
                for o in ARRIVE_ORDER
            ]

            if l + 1 < N_LAYERS:
                for c in w_copies(l + 1):
                    c.wait()
                winb_vm[(l + 1) % 2] = win_vm[(l + 1) % 2].astype(jnp.bfloat16)
                woutb_vm[(l + 1) % 2] = wout_vm[(l + 1) % 2].astype(jnp.bfloat16)

            for r in pending:
                r.wait_send()
            pending = new_sends

            for o in ARRIVE_ORDER:
                s = (my + o) % N_DEV
                wait_recv_from(p_rs, o, rs_recv.at[l, s])
            psum = psbuf[pl.ds(my * m_per, m_per), :].astype(jnp.float32)
            for o in range(1, N_DEV):
                s = (my + o) % N_DEV
                psum = psum + rs_recv[l, s].astype(jnp.float32)

            if not last:
                dst = acts.at[l + 1]
                dst[pl.ds(my * m_per, m_per), :] = psum.astype(jnp.bfloat16)
                for o in AG_ORDER:
                    pending.append(
                        send_one(p_ag, o, rows(dst, my), rows(dst, my)))
            else:
                outstage[...] = psum
                pltpu.make_async_copy(
                    outstage, rows(out_ref, my), outcopy_sem.at[0]).start()
                for o in AG_ORDER:
                    pending.append(
                        send_one(p_ag, o, outstage, rows(out_ref, my)))

        for o in ARRIVE_ORDER:
            s = (my + o) % N_DEV
            wait_recv_from(2 * N_LAYERS, o, rows(out_ref, s))
        pltpu.make_async_copy(
            outstage, rows(out_ref, my), outcopy_sem.at[0]).wait()
        for r in pending:
            r.wait_send()

    any_spec = pl.BlockSpec(memory_space=pl.ANY)
    vmem_spec = pl.BlockSpec(memory_space=pltpu.VMEM)
    return pl.pallas_call(
        body,
        out_shape=jax.ShapeDtypeStruct((M, d), jnp.float32),
        in_specs=[vmem_spec] + [any_spec] * 6,
        out_specs=pl.BlockSpec(memory_space=pl.ANY),
        scratch_shapes=[
            pltpu.VMEM((N_LAYERS, M, d), jnp.bfloat16),
            pltpu.VMEM((N_LAYERS, N_DEV, m_per, d), jnp.bfloat16),
            pltpu.VMEM((M, d), jnp.bfloat16),
            pltpu.VMEM((2, d, 2 * d), jnp.float32),
            pltpu.VMEM((2, 2 * d, d), jnp.float32),
            pltpu.VMEM((2, d, 2 * d), jnp.bfloat16),
            pltpu.VMEM((2, 2 * d, d), jnp.bfloat16),
            pltpu.VMEM((m_per, d), jnp.float32),
            pltpu.SemaphoreType.DMA((N_PHASES, N_DEV - 1)),
            pltpu.SemaphoreType.DMA((N_PHASES, N_DEV)),
            pltpu.SemaphoreType.DMA((N_LAYERS, 2)),
            pltpu.SemaphoreType.DMA((1,)),
        ],
        compiler_params=pltpu.CompilerParams(
            collective_id=0,
            vmem_limit_bytes=100 * 1024 * 1024,
        ),
    )(x, Win0, Wout0, Win1, Wout1, Win2, Wout2)
